# baseline (speedup 1.0000x reference)
import jax
import jax.numpy as jnp
from jax.experimental import pallas as pl

G = 16
M = 3


def _mm_body(x_ref, w_ref, o_ref):
    o_ref[...] = jnp.dot(x_ref[...], w_ref[...],
                         preferred_element_type=jnp.float32)


def _pallas_matmul(x, w, block_rows=512):
    N, K = x.shape
    K2, C = w.shape
    assert K == K2
    Np = (N + block_rows - 1) // block_rows * block_rows
    Kp = (K + 127) // 128 * 128
    Cp = (C + 127) // 128 * 128
    xp = jnp.pad(x, ((0, Np - N), (0, Kp - K)))
    wp = jnp.pad(w, ((0, Kp - K), (0, Cp - C)))
    out = pl.pallas_call(
        _mm_body,
        grid=(Np // block_rows,),
        in_specs=[
            pl.BlockSpec((block_rows, Kp), lambda i: (i, 0)),
            pl.BlockSpec((Kp, Cp), lambda i: (0, 0)),
        ],
        out_specs=pl.BlockSpec((block_rows, Cp), lambda i: (i, 0)),
        out_shape=jax.ShapeDtypeStruct((Np, Cp), jnp.float32),
    )(xp, wp)
    return out[:N, :C]


def _corner_tables(pseudo):
    v = jnp.clip(pseudo, 0.0, 1.0) * (M - 1)
    lo = jnp.clip(jnp.floor(v), 0, M - 2)
    fr = v - lo
    lo = lo.astype(jnp.int32)
    offs = jnp.array([[(c >> 0) & 1, (c >> 1) & 1, (c >> 2) & 1]
                      for c in range(8)], dtype=jnp.int32)
    cidx = ((lo[:, None, 0] + offs[None, :, 0])
            + M * (lo[:, None, 1] + offs[None, :, 1])
            + M * M * (lo[:, None, 2] + offs[None, :, 2]))
    w0 = jnp.where(offs[None, :, 0] == 1, fr[:, None, 0], 1.0 - fr[:, None, 0])
    w1 = jnp.where(offs[None, :, 1] == 1, fr[:, None, 1], 1.0 - fr[:, None, 1])
    w2 = jnp.where(offs[None, :, 2] == 1, fr[:, None, 2], 1.0 - fr[:, None, 2])
    return cidx, w0 * w1 * w2


def _conv(Z, row, col, cidx, w8, Cout, bias):
    N = Z.shape[0]
    E = row.shape[0]
    Zr = Z[:, :27 * Cout].reshape(N * 27, Cout)
    root_term = Z[:, 27 * Cout:]
    mask = (row != col)
    maskf = mask.astype(jnp.float32)
    fidx = row[:, None] * 27 + cidx
    g = jnp.take(Zr, fidx.reshape(-1), axis=0).reshape(E, 8, Cout)
    msg = jnp.einsum('ec,eco->eo', w8, g) * maskf[:, None]
    payload = jnp.concatenate(
        [msg, maskf[:, None], jnp.ones((E, 1), jnp.float32)], axis=1)
    seg = jax.ops.segment_sum(payload, col, num_segments=N)
    agg = seg[:, :Cout] / jnp.maximum(seg[:, Cout], 1.0)[:, None]
    degc = seg[:, Cout + 1]
    return jax.nn.elu(agg + root_term + bias), degc


def _edge_weight(degc, d, row, col):
    inv = jnp.where(degc > 0, 1.0 / jnp.maximum(degc, 1.0), 0.0)
    return d * (jnp.take(inv, row) + jnp.take(inv, col))


def _graclus(row, col, weight, N):
    mask = row != col
    wm = jnp.where(mask, weight, -jnp.inf)
    wmax = jax.ops.segment_max(wm, row, num_segments=N)
    isbest = mask & (wm >= jnp.take(wmax, row)) & jnp.isfinite(wm)
    cand = jnp.where(isbest, col, -1)
    best = jax.ops.segment_max(cand, row, num_segments=N)
    best = jnp.maximum(best, -1)
    safe = jnp.clip(best, 0, N - 1)
    idx = jnp.arange(N)
    mutual = (best >= 0) & (jnp.take(best, safe) == idx)
    return jnp.where(mutual, jnp.minimum(idx, safe), idx)


def kernel(x, edge_index, edge_attr, pos, batch, W1, root1, b1, W2, root2, b2,
           fc1_w, fc1_b, fc2_w, fc2_b):
    N = x.shape[0]
    E = edge_index.shape[1]
    row, col = edge_index[0], edge_index[1]
    cidx, w8 = _corner_tables(edge_attr)

    Wf1 = jnp.transpose(W1, (1, 0, 2)).reshape(x.shape[1], 27 * 8)
    Z1 = _pallas_matmul(x, jnp.concatenate([Wf1, root1], axis=1))
    h, degc1 = _conv(Z1, row, col, cidx, w8, 8, b1)

    diff = jnp.take(pos, row, axis=0) - jnp.take(pos, col, axis=0)
    d = jnp.sqrt(jnp.maximum(jnp.sum(diff * diff, axis=1), 1e-12))
    w = _edge_weight(degc1, d, row, col)
    c1 = _graclus(row, col, w, N)

    batchf = batch.astype(jnp.float32)
    sum_pay = jnp.concatenate([pos, jnp.ones((N, 1), jnp.float32)], axis=1)
    sums1 = jax.ops.segment_sum(sum_pay, c1, num_segments=N)
    max_pay = jnp.concatenate([h, batchf[:, None]], axis=1)
    maxs1 = jax.ops.segment_max(max_pay, c1, num_segments=N)
    cnt1 = sums1[:, 3]
    nonempty1 = cnt1 > 0
    h2 = jnp.where(nonempty1[:, None], maxs1[:, :8], 0.0)
    pos2 = sums1[:, :3] / jnp.maximum(cnt1, 1.0)[:, None]
    batch2 = jnp.where(nonempty1, maxs1[:, 8], float(G))

    row2, col2 = jnp.take(c1, row), jnp.take(c1, col)
    Wf2 = jnp.transpose(W2, (1, 0, 2)).reshape(8, 27 * 16)
    Z2 = _pallas_matmul(h2, jnp.concatenate([Wf2, root2], axis=1))
    h3, degc2 = _conv(Z2, row2, col2, cidx, w8, 16, b2)

    diff2 = jnp.take(pos2, row2, axis=0) - jnp.take(pos2, col2, axis=0)
    d2 = jnp.sqrt(jnp.maximum(jnp.sum(diff2 * diff2, axis=1), 1e-12))
    w2 = _edge_weight(degc2, d2, row2, col2)
    c2 = _graclus(row2, col2, w2, N)

    cnt2 = jax.ops.segment_sum(jnp.ones((N,), jnp.float32), c2, num_segments=N)
    max_pay2 = jnp.concatenate([h3, batch2[:, None]], axis=1)
    maxs2 = jax.ops.segment_max(max_pay2, c2, num_segments=N)
    nonempty2 = cnt2 > 0
    xp = jnp.where(nonempty2[:, None], maxs2[:, :16], 0.0)
    bp = jnp.where(nonempty2, maxs2[:, 16], float(G)).astype(jnp.int32)
    gpay = jnp.concatenate([xp, jnp.ones((N, 1), jnp.float32)], axis=1)
    gsum = jax.ops.segment_sum(gpay, bp, num_segments=G + 1)
    gx = gsum[:G, :16] / jnp.maximum(gsum[:G, 16], 1.0)[:, None]

    out = jax.nn.elu(gx @ fc1_w + fc1_b)
    out = jax.nn.elu(out @ fc2_w + fc2_b)
    return out


# speedup vs baseline: 4.9136x; 4.9136x over previous
import jax
import jax.numpy as jnp
from jax.experimental import pallas as pl

G = 16
M = 3


def _mm_body(x_ref, w_ref, o_ref):
    o_ref[...] = jnp.dot(x_ref[...], w_ref[...],
                         preferred_element_type=jnp.float32)


def _pallas_matmul(x, w, block_rows=512):
    N, K = x.shape
    K2, C = w.shape
    assert K == K2
    Np = (N + block_rows - 1) // block_rows * block_rows
    Kp = (K + 127) // 128 * 128
    Cp = (C + 127) // 128 * 128
    xp = jnp.pad(x, ((0, Np - N), (0, Kp - K)))
    wp = jnp.pad(w, ((0, Kp - K), (0, Cp - C)))
    out = pl.pallas_call(
        _mm_body,
        grid=(Np // block_rows,),
        in_specs=[
            pl.BlockSpec((block_rows, Kp), lambda i: (i, 0)),
            pl.BlockSpec((Kp, Cp), lambda i: (0, 0)),
        ],
        out_specs=pl.BlockSpec((block_rows, Cp), lambda i: (i, 0)),
        out_shape=jax.ShapeDtypeStruct((Np, Cp), jnp.float32),
    )(xp, wp)
    return out[:N, :C]


def _corner_tables(pseudo):
    v = jnp.clip(pseudo, 0.0, 1.0) * (M - 1)
    lo = jnp.clip(jnp.floor(v), 0, M - 2)
    fr = v - lo
    lo = lo.astype(jnp.int32)
    offs = jnp.array([[(c >> 0) & 1, (c >> 1) & 1, (c >> 2) & 1]
                      for c in range(8)], dtype=jnp.int32)
    cidx = ((lo[:, None, 0] + offs[None, :, 0])
            + M * (lo[:, None, 1] + offs[None, :, 1])
            + M * M * (lo[:, None, 2] + offs[None, :, 2]))
    w0 = jnp.where(offs[None, :, 0] == 1, fr[:, None, 0], 1.0 - fr[:, None, 0])
    w1 = jnp.where(offs[None, :, 1] == 1, fr[:, None, 1], 1.0 - fr[:, None, 1])
    w2 = jnp.where(offs[None, :, 2] == 1, fr[:, None, 2], 1.0 - fr[:, None, 2])
    return cidx, w0 * w1 * w2


def _conv(Z, row, col, cidx, w8, Cout, bias):
    N = Z.shape[0]
    E = row.shape[0]
    root_term = Z[:, 27 * Cout:]
    mask = (row != col)
    maskf = mask.astype(jnp.float32)
    B = jnp.sum(w8[:, :, None]
                * (cidx[:, :, None] == jnp.arange(27)[None, None, :]),
                axis=1)
    y = jnp.take(Z[:, :27 * Cout], row, axis=0).reshape(E, 27, Cout)
    msg = jnp.sum(B[:, :, None] * y, axis=1) * maskf[:, None]
    payload = jnp.concatenate(
        [msg, maskf[:, None], jnp.ones((E, 1), jnp.float32)], axis=1)
    seg = jax.ops.segment_sum(payload, col, num_segments=N)
    agg = seg[:, :Cout] / jnp.maximum(seg[:, Cout], 1.0)[:, None]
    degc = seg[:, Cout + 1]
    return jax.nn.elu(agg + root_term + bias), degc


def _edge_weight(degc, d, row, col):
    inv = jnp.where(degc > 0, 1.0 / jnp.maximum(degc, 1.0), 0.0)
    return d * (jnp.take(inv, row) + jnp.take(inv, col))


def _graclus(row, col, weight, N):
    mask = row != col
    wm = jnp.where(mask, weight, -jnp.inf)
    wmax = jax.ops.segment_max(wm, row, num_segments=N)
    isbest = mask & (wm >= jnp.take(wmax, row)) & jnp.isfinite(wm)
    cand = jnp.where(isbest, col, -1)
    best = jax.ops.segment_max(cand, row, num_segments=N)
    best = jnp.maximum(best, -1)
    safe = jnp.clip(best, 0, N - 1)
    idx = jnp.arange(N)
    mutual = (best >= 0) & (jnp.take(best, safe) == idx)
    return jnp.where(mutual, jnp.minimum(idx, safe), idx)


def kernel(x, edge_index, edge_attr, pos, batch, W1, root1, b1, W2, root2, b2,
           fc1_w, fc1_b, fc2_w, fc2_b):
    N = x.shape[0]
    E = edge_index.shape[1]
    row, col = edge_index[0], edge_index[1]
    cidx, w8 = _corner_tables(edge_attr)

    Wf1 = jnp.transpose(W1, (1, 0, 2)).reshape(x.shape[1], 27 * 8)
    Z1 = _pallas_matmul(x, jnp.concatenate([Wf1, root1], axis=1))
    h, degc1 = _conv(Z1, row, col, cidx, w8, 8, b1)

    diff = jnp.take(pos, row, axis=0) - jnp.take(pos, col, axis=0)
    d = jnp.sqrt(jnp.maximum(jnp.sum(diff * diff, axis=1), 1e-12))
    w = _edge_weight(degc1, d, row, col)
    c1 = _graclus(row, col, w, N)

    batchf = batch.astype(jnp.float32)
    sum_pay = jnp.concatenate([pos, jnp.ones((N, 1), jnp.float32)], axis=1)
    sums1 = jax.ops.segment_sum(sum_pay, c1, num_segments=N)
    max_pay = jnp.concatenate([h, batchf[:, None]], axis=1)
    maxs1 = jax.ops.segment_max(max_pay, c1, num_segments=N)
    cnt1 = sums1[:, 3]
    nonempty1 = cnt1 > 0
    h2 = jnp.where(nonempty1[:, None], maxs1[:, :8], 0.0)
    pos2 = sums1[:, :3] / jnp.maximum(cnt1, 1.0)[:, None]
    batch2 = jnp.where(nonempty1, maxs1[:, 8], float(G))

    row2, col2 = jnp.take(c1, row), jnp.take(c1, col)
    Wf2 = jnp.transpose(W2, (1, 0, 2)).reshape(8, 27 * 16)
    Z2 = _pallas_matmul(h2, jnp.concatenate([Wf2, root2], axis=1))
    h3, degc2 = _conv(Z2, row2, col2, cidx, w8, 16, b2)

    diff2 = jnp.take(pos2, row2, axis=0) - jnp.take(pos2, col2, axis=0)
    d2 = jnp.sqrt(jnp.maximum(jnp.sum(diff2 * diff2, axis=1), 1e-12))
    w2 = _edge_weight(degc2, d2, row2, col2)
    c2 = _graclus(row2, col2, w2, N)

    cnt2 = jax.ops.segment_sum(jnp.ones((N,), jnp.float32), c2, num_segments=N)
    max_pay2 = jnp.concatenate([h3, batch2[:, None]], axis=1)
    maxs2 = jax.ops.segment_max(max_pay2, c2, num_segments=N)
    nonempty2 = cnt2 > 0
    xp = jnp.where(nonempty2[:, None], maxs2[:, :16], 0.0)
    bp = jnp.where(nonempty2, maxs2[:, 16], float(G)).astype(jnp.int32)
    gpay = jnp.concatenate([xp, jnp.ones((N, 1), jnp.float32)], axis=1)
    gsum = jax.ops.segment_sum(gpay, bp, num_segments=G + 1)
    gx = gsum[:G, :16] / jnp.maximum(gsum[:G, 16], 1.0)[:, None]

    out = jax.nn.elu(gx @ fc1_w + fc1_b)
    out = jax.nn.elu(out @ fc2_w + fc2_b)
    return out
